# baseline (speedup 1.0000x reference)
import functools

import jax
import jax.numpy as jnp
from jax import lax
from jax.experimental import pallas as pl
from jax.experimental.pallas import tpu as pltpu
from jax.experimental.pallas import tpu_sc as plsc

B, S, D, E, TOP_K = 4, 4096, 4096, 64, 8
T = B * S

BT = 1024
CHUNK_TOKENS = (12288, 4096)

NC, NS, L = 2, 16, 16
NW = NC * NS


def _logits_body(x_ref, m_ref, w_ref, b_ref, lg_ref):
    lg = lax.dot_general(
        w_ref[...], x_ref[...],
        dimension_numbers=(((1,), (1,)), ((), ())),
        preferred_element_type=jnp.float32,
    )
    lg = lg + b_ref[...]
    lg_ref[...] = jnp.where(m_ref[...] != 1, -jnp.inf, lg)


def _make_route(tc):
    tok_w = tc // NW
    ng = tok_w // L

    def _route_body(lg_hbm, ew_hbm, ei_hbm, lg_v, ew_v, ei_v, sem):
        wid = lax.axis_index("s") * NC + lax.axis_index("c")
        base = wid * tok_w
        pltpu.sync_copy(lg_hbm.at[:, pl.ds(base, tok_w)], lg_v)

        neg_inf = jnp.full((L,), -jnp.inf, jnp.float32)

        def group(g, _):
            def expert(e, carry):
                topv = list(carry[:TOP_K])
                topi = list(carry[TOP_K:])
                xv = lg_v[e, pl.ds(g * L, L)]
                xi = jnp.full((L,), e, jnp.int32)
                for j in range(TOP_K):
                    c = xv > topv[j]
                    nv = jnp.where(c, xv, topv[j])
                    xv = jnp.where(c, topv[j], xv)
                    ni = jnp.where(c, xi, topi[j])
                    xi = jnp.where(c, topi[j], xi)
                    topv[j] = nv
                    topi[j] = ni
                return tuple(topv) + tuple(topi)

            init = (neg_inf,) * TOP_K + (jnp.zeros((L,), jnp.int32),) * TOP_K
            carry = lax.fori_loop(0, E, expert, init)
            topv = list(carry[:TOP_K])
            topi = list(carry[TOP_K:])
            es = [jnp.exp(v - topv[0]) for v in topv]
            tot = es[0]
            for v in es[1:]:
                tot = tot + v
            for j in range(TOP_K):
                ew_v[j, pl.ds(g * L, L)] = es[j] / tot
                ei_v[j, pl.ds(g * L, L)] = topi[j]
            return 0

        lax.fori_loop(0, ng, group, 0)
        pltpu.sync_copy(ew_v, ew_hbm.at[:, pl.ds(base, tok_w)])
        pltpu.sync_copy(ei_v, ei_hbm.at[:, pl.ds(base, tok_w)])

    return functools.partial(
        pl.kernel,
        out_type=[
            jax.ShapeDtypeStruct((TOP_K, tc), jnp.float32),
            jax.ShapeDtypeStruct((TOP_K, tc), jnp.int32),
        ],
        mesh=plsc.VectorSubcoreMesh(core_axis_name="c", subcore_axis_name="s"),
        scratch_types=[
            pltpu.VMEM((E, tok_w), jnp.float32),
            pltpu.VMEM((TOP_K, tok_w), jnp.float32),
            pltpu.VMEM((TOP_K, tok_w), jnp.int32),
            pltpu.SemaphoreType.DMA,
        ],
    )(_route_body)


_routes = {tc: _make_route(tc) for tc in set(CHUNK_TOKENS)}


@jax.jit
def kernel(x, attention_mask, W, b):
    x2 = x.reshape(T, D)
    m2 = attention_mask.reshape(1, T)
    b2 = b.reshape(E, 1)

    off = 0
    ews, eis = [], []
    for tc in CHUNK_TOKENS:
        bt = 1536 if tc % 1536 == 0 else BT
        nblk = tc // bt
        blk0 = off // bt
        logits = pl.pallas_call(
            _logits_body,
            grid=(nblk,),
            in_specs=[
                pl.BlockSpec((bt, D), lambda i, blk0=blk0: (blk0 + i, 0)),
                pl.BlockSpec((1, bt), lambda i, blk0=blk0: (0, blk0 + i)),
                pl.BlockSpec((E, D), lambda i: (0, 0)),
                pl.BlockSpec((E, 1), lambda i: (0, 0)),
            ],
            out_specs=pl.BlockSpec((E, bt), lambda i: (0, i)),
            out_shape=jax.ShapeDtypeStruct((E, tc), jnp.float32),
        )(x2, m2, W, b2)
        ew_c, ei_c = _routes[tc](logits)
        ews.append(ew_c)
        eis.append(ei_c)
        off += tc

    ew = jnp.concatenate(ews, axis=1)
    ei = jnp.concatenate(eis, axis=1)
    return (
        ew.T.reshape(B, S, TOP_K),
        ei.T.reshape(B, S, TOP_K),
    )


# speedup vs baseline: 1.0192x; 1.0192x over previous
import functools

import jax
import jax.numpy as jnp
from jax import lax
from jax.experimental import pallas as pl
from jax.experimental.pallas import tpu as pltpu
from jax.experimental.pallas import tpu_sc as plsc

B, S, D, E, TOP_K = 4, 4096, 4096, 64, 8
T = B * S

BT = 1024
CHUNK_TOKENS = (12288, 4096)

NC, NS, L = 2, 16, 16
NW = NC * NS


def _logits_body(x_ref, m_ref, w_ref, b_ref, lg_ref):
    lg = lax.dot_general(
        w_ref[...], x_ref[...],
        dimension_numbers=(((1,), (1,)), ((), ())),
        preferred_element_type=jnp.float32,
    )
    lg = lg + b_ref[...]
    lg_ref[...] = jnp.where(m_ref[...] != 1, -jnp.inf, lg)


def _make_route(tc):
    tok_w = tc // NW
    ng = tok_w // L

    def _route_body(lg_hbm, ew_hbm, ei_hbm, lg_v, ew_v, ei_v, sem):
        wid = lax.axis_index("s") * NC + lax.axis_index("c")
        base = wid * tok_w
        pltpu.sync_copy(lg_hbm.at[:, pl.ds(base, tok_w)], lg_v)

        neg_inf = jnp.full((L,), -jnp.inf, jnp.float32)

        zero_i = jnp.zeros((L,), jnp.int32)

        def group(g, _):
            def expert(e, carry):
                out = []
                for half in range(2):
                    topv = list(carry[half * 2 * TOP_K: half * 2 * TOP_K + TOP_K])
                    topi = list(carry[half * 2 * TOP_K + TOP_K: (half + 1) * 2 * TOP_K])
                    eh = e + half * (E // 2)
                    xv = lg_v[eh, pl.ds(g * L, L)]
                    xi = jnp.full((L,), eh, jnp.int32)
                    for j in range(TOP_K):
                        c = xv > topv[j]
                        nv = jnp.where(c, xv, topv[j])
                        xv = jnp.where(c, topv[j], xv)
                        ni = jnp.where(c, xi, topi[j])
                        xi = jnp.where(c, topi[j], xi)
                        topv[j] = nv
                        topi[j] = ni
                    out.extend(topv)
                    out.extend(topi)
                return tuple(out)

            init = ((neg_inf,) * TOP_K + (zero_i,) * TOP_K) * 2
            carry = lax.fori_loop(0, E // 2, expert, init)
            av = list(carry[0:TOP_K])
            ai = list(carry[TOP_K:2 * TOP_K])
            bv = list(carry[2 * TOP_K:3 * TOP_K])
            bi = list(carry[3 * TOP_K:4 * TOP_K])

            cv, ci = [], []
            for i in range(TOP_K):
                c = av[i] >= bv[TOP_K - 1 - i]
                cv.append(jnp.where(c, av[i], bv[TOP_K - 1 - i]))
                ci.append(jnp.where(c, ai[i], bi[TOP_K - 1 - i]))
            for dist in (4, 2, 1):
                pairs = [i for i in range(TOP_K) if (i % (2 * dist)) < dist and i + dist < TOP_K]
                for i in pairs:
                    c = cv[i] >= cv[i + dist]
                    hv = jnp.where(c, cv[i], cv[i + dist])
                    lv = jnp.where(c, cv[i + dist], cv[i])
                    hi_ = jnp.where(c, ci[i], ci[i + dist])
                    li_ = jnp.where(c, ci[i + dist], ci[i])
                    cv[i], cv[i + dist] = hv, lv
                    ci[i], ci[i + dist] = hi_, li_
            topv = cv
            topi = ci
            es = [jnp.exp(v - topv[0]) for v in topv]
            tot = es[0]
            for v in es[1:]:
                tot = tot + v
            for j in range(TOP_K):
                ew_v[j, pl.ds(g * L, L)] = es[j] / tot
                ei_v[j, pl.ds(g * L, L)] = topi[j]
            return 0

        lax.fori_loop(0, ng, group, 0)
        pltpu.sync_copy(ew_v, ew_hbm.at[:, pl.ds(base, tok_w)])
        pltpu.sync_copy(ei_v, ei_hbm.at[:, pl.ds(base, tok_w)])

    return functools.partial(
        pl.kernel,
        out_type=[
            jax.ShapeDtypeStruct((TOP_K, tc), jnp.float32),
            jax.ShapeDtypeStruct((TOP_K, tc), jnp.int32),
        ],
        mesh=plsc.VectorSubcoreMesh(core_axis_name="c", subcore_axis_name="s"),
        scratch_types=[
            pltpu.VMEM((E, tok_w), jnp.float32),
            pltpu.VMEM((TOP_K, tok_w), jnp.float32),
            pltpu.VMEM((TOP_K, tok_w), jnp.int32),
            pltpu.SemaphoreType.DMA,
        ],
    )(_route_body)


_routes = {tc: _make_route(tc) for tc in set(CHUNK_TOKENS)}


@jax.jit
def kernel(x, attention_mask, W, b):
    x2 = x.reshape(T, D)
    m2 = attention_mask.reshape(1, T)
    b2 = b.reshape(E, 1)

    off = 0
    ews, eis = [], []
    for tc in CHUNK_TOKENS:
        bt = BT
        nblk = tc // bt
        blk0 = off // bt
        logits = pl.pallas_call(
            _logits_body,
            grid=(nblk,),
            in_specs=[
                pl.BlockSpec((bt, D), lambda i, blk0=blk0: (blk0 + i, 0)),
                pl.BlockSpec((1, bt), lambda i, blk0=blk0: (0, blk0 + i)),
                pl.BlockSpec((E, D), lambda i: (0, 0)),
                pl.BlockSpec((E, 1), lambda i: (0, 0)),
            ],
            out_specs=pl.BlockSpec((E, bt), lambda i: (0, i)),
            out_shape=jax.ShapeDtypeStruct((E, tc), jnp.float32),
        )(x2, m2, W, b2)
        ew_c, ei_c = _routes[tc](logits)
        ews.append(ew_c)
        eis.append(ei_c)
        off += tc

    ew = jnp.concatenate(ews, axis=1)
    ei = jnp.concatenate(eis, axis=1)
    return (
        ew.T.reshape(B, S, TOP_K),
        ei.T.reshape(B, S, TOP_K),
    )


# speedup vs baseline: 1.0238x; 1.0045x over previous
import functools

import jax
import jax.numpy as jnp
from jax import lax
from jax.experimental import pallas as pl
from jax.experimental.pallas import tpu as pltpu
from jax.experimental.pallas import tpu_sc as plsc

B, S, D, E, TOP_K = 4, 4096, 4096, 64, 8
T = B * S

BT = 1024
CHUNK_TOKENS = (12288, 4096)

NC, NS, L = 2, 16, 16
NW = NC * NS


def _logits_body(x_ref, m_ref, w_ref, b_ref, lg_ref):
    lg = lax.dot_general(
        w_ref[...], x_ref[...],
        dimension_numbers=(((1,), (1,)), ((), ())),
        preferred_element_type=jnp.float32,
    )
    lg = lg + b_ref[...]
    lg_ref[...] = jnp.where(m_ref[...] != 1, -jnp.inf, lg)


def _make_route(tc):
    tok_w = tc // NW
    ng = tok_w // L

    def _route_body(lg_hbm, ew_hbm, ei_hbm, lg_v, ew_v, ei_v, sem):
        wid = lax.axis_index("s") * NC + lax.axis_index("c")
        base = wid * tok_w
        pltpu.sync_copy(lg_hbm.at[:, pl.ds(base, tok_w)], lg_v)

        neg_inf = jnp.full((L,), -jnp.inf, jnp.float32)

        def group(g, _):
            def expert(e, carry):
                topv = list(carry[:TOP_K])
                topi = list(carry[TOP_K:])
                xv = lg_v[e, pl.ds(g * L, L)]
                xi = jnp.full((L,), e, jnp.int32)
                for j in range(TOP_K):
                    c = xv > topv[j]
                    nv = jnp.where(c, xv, topv[j])
                    xv = jnp.where(c, topv[j], xv)
                    ni = jnp.where(c, xi, topi[j])
                    xi = jnp.where(c, topi[j], xi)
                    topv[j] = nv
                    topi[j] = ni
                return tuple(topv) + tuple(topi)

            init = (neg_inf,) * TOP_K + (jnp.zeros((L,), jnp.int32),) * TOP_K
            carry = lax.fori_loop(0, E, expert, init)
            topv = list(carry[:TOP_K])
            topi = list(carry[TOP_K:])
            es = [jnp.exp(v - topv[0]) for v in topv]
            tot = es[0]
            for v in es[1:]:
                tot = tot + v
            for j in range(TOP_K):
                ew_v[j, pl.ds(g * L, L)] = es[j] / tot
                ei_v[j, pl.ds(g * L, L)] = topi[j]
            return 0

        lax.fori_loop(0, ng, group, 0)
        pltpu.sync_copy(ew_v, ew_hbm.at[:, pl.ds(base, tok_w)])
        pltpu.sync_copy(ei_v, ei_hbm.at[:, pl.ds(base, tok_w)])

    return functools.partial(
        pl.kernel,
        out_type=[
            jax.ShapeDtypeStruct((TOP_K, tc), jnp.float32),
            jax.ShapeDtypeStruct((TOP_K, tc), jnp.int32),
        ],
        mesh=plsc.VectorSubcoreMesh(core_axis_name="c", subcore_axis_name="s"),
        scratch_types=[
            pltpu.VMEM((E, tok_w), jnp.float32),
            pltpu.VMEM((TOP_K, tok_w), jnp.float32),
            pltpu.VMEM((TOP_K, tok_w), jnp.int32),
            pltpu.SemaphoreType.DMA,
        ],
    )(_route_body)


_routes = {tc: _make_route(tc) for tc in set(CHUNK_TOKENS)}


@jax.jit
def kernel(x, attention_mask, W, b):
    x2 = x.reshape(T, D)
    m2 = attention_mask.reshape(1, T)
    b2 = b.reshape(E, 1)

    off = 0
    ews, eis = [], []
    for tc in CHUNK_TOKENS:
        nblk = tc // BT
        blk0 = off // BT
        logits = pl.pallas_call(
            _logits_body,
            grid=(nblk,),
            in_specs=[
                pl.BlockSpec((BT, D), lambda i, blk0=blk0: (blk0 + i, 0)),
                pl.BlockSpec((1, BT), lambda i, blk0=blk0: (0, blk0 + i)),
                pl.BlockSpec((E, D), lambda i: (0, 0)),
                pl.BlockSpec((E, 1), lambda i: (0, 0)),
            ],
            out_specs=pl.BlockSpec((E, BT), lambda i: (0, i)),
            out_shape=jax.ShapeDtypeStruct((E, tc), jnp.float32),
        )(x2, m2, W, b2)
        ew_c, ei_c = _routes[tc](logits)
        ews.append(ew_c)
        eis.append(ei_c)
        off += tc

    ew = jnp.concatenate(ews, axis=1)
    ei = jnp.concatenate(eis, axis=1)
    return (
        ew.T.reshape(B, S, TOP_K),
        ei.T.reshape(B, S, TOP_K),
    )
